# baseline (speedup 1.0000x reference)
import jax
import jax.numpy as jnp
from jax.experimental import pallas as pl
from jax.experimental.pallas import tpu as pltpu

B, S, D = 2, 2048, 1024
HR, HE, HF = 256, 4096, 1024

BS = 512
NS = S // BS
HEB = 256
NK = HE // HEB


def _router_kernel(x_ref, wr1_ref, br1_ref, wdom_ref, wgate_ref,
                   dlog_ref, glog_ref):
    pooled = jnp.mean(x_ref[...], axis=1)
    h = jnp.tanh(
        jnp.dot(pooled, wr1_ref[...], preferred_element_type=jnp.float32)
        + br1_ref[...])
    dlog_ref[...] = jnp.dot(h, wdom_ref[...],
                            preferred_element_type=jnp.float32)
    glog_ref[...] = jnp.dot(h, wgate_ref[...],
                            preferred_element_type=jnp.float32)


def _expert_kernel(dom_ref, conf_ref, x_ref, w1m_ref, w1l_ref, b1_ref,
                   w2m_ref, w2l_ref, b2_ref, wf1_ref, bf1_ref, wf2_ref,
                   bf2_ref, o_ref):
    b = pl.program_id(0)
    k = pl.program_id(1)
    d = dom_ref[b]

    def _expert_step(w1_ref, w2_ref):
        w1 = w1_ref[...]
        w2 = w2_ref[...]
        for si in range(NS):
            sl = pl.ds(si * BS, BS)
            h = jax.nn.gelu(
                jnp.dot(x_ref[0, sl, :], w1,
                        preferred_element_type=jnp.float32)
                + b1_ref[0])
            part = jnp.dot(h, w2, preferred_element_type=jnp.float32)

            @pl.when(k == 0)
            def _():
                o_ref[0, sl, :] = part

            @pl.when(k > 0)
            def _():
                o_ref[0, sl, :] += part

    @pl.when(d == 0)
    def _():
        _expert_step(w1m_ref, w2m_ref)

    @pl.when(d != 0)
    def _():
        _expert_step(w1l_ref, w2l_ref)

    @pl.when(k == NK - 1)
    def _():
        c = conf_ref[b]
        for si in range(NS):
            sl = pl.ds(si * BS, BS)
            e = o_ref[0, sl, :] + b2_ref[0]
            t = jnp.tanh(
                jnp.dot(e, wf1_ref[...], preferred_element_type=jnp.float32)
                + bf1_ref[...])
            f = jnp.dot(t, wf2_ref[...],
                        preferred_element_type=jnp.float32) + bf2_ref[...]
            o_ref[0, sl, :] = c * f + x_ref[0, sl, :]


@jax.jit
def kernel(x, Wr1, br1, Wdom, Wop, Wtask, Wgate, Wm1, bm1, Wm2, bm2,
           Wl1, bl1, Wl2, bl2, Wf1, bf1, Wf2, bf2):
    del Wop, Wtask

    dlog, glog = pl.pallas_call(
        _router_kernel,
        out_shape=(
            jax.ShapeDtypeStruct((B, 2), jnp.float32),
            jax.ShapeDtypeStruct((B, 2), jnp.float32),
        ),
    )(x, Wr1, br1.reshape(1, HR), Wdom, Wgate)

    dom = (dlog[:, 1] > dlog[:, 0]).astype(jnp.int32)
    gmax = jnp.max(glog, axis=1, keepdims=True)
    eg = jnp.exp(glog - gmax)
    conf = jnp.take_along_axis(eg, dom[:, None], axis=1)[:, 0] / jnp.sum(eg, axis=1)

    b1s = jnp.stack([bm1, bl1]).reshape(2, 1, HE)
    b2s = jnp.stack([bm2, bl2]).reshape(2, 1, D)

    def w1_idx(sel):
        def idx(b, k, dom, conf):
            return (0, jnp.where(dom[b] == sel, k, 0))
        return idx

    def w2_idx(sel):
        def idx(b, k, dom, conf):
            return (jnp.where(dom[b] == sel, k, 0), 0)
        return idx

    grid = (B, NK)
    out = pl.pallas_call(
        _expert_kernel,
        grid_spec=pltpu.PrefetchScalarGridSpec(
            num_scalar_prefetch=2,
            grid=grid,
            in_specs=[
                pl.BlockSpec((1, S, D), lambda b, k, dom, conf: (b, 0, 0)),
                pl.BlockSpec((D, HEB), w1_idx(0)),
                pl.BlockSpec((D, HEB), w1_idx(1)),
                pl.BlockSpec((1, 1, HEB),
                             lambda b, k, dom, conf: (dom[b], 0, k)),
                pl.BlockSpec((HEB, D), w2_idx(0)),
                pl.BlockSpec((HEB, D), w2_idx(1)),
                pl.BlockSpec((1, 1, D),
                             lambda b, k, dom, conf: (dom[b], 0, 0)),
                pl.BlockSpec((D, HF), lambda b, k, dom, conf: (0, 0)),
                pl.BlockSpec((1, HF), lambda b, k, dom, conf: (0, 0)),
                pl.BlockSpec((HF, D), lambda b, k, dom, conf: (0, 0)),
                pl.BlockSpec((1, D), lambda b, k, dom, conf: (0, 0)),
            ],
            out_specs=pl.BlockSpec((1, S, D),
                                   lambda b, k, dom, conf: (b, 0, 0)),
        ),
        out_shape=jax.ShapeDtypeStruct((B, S, D), jnp.float32),
    )(dom, conf, x, Wm1, Wl1, b1s, Wm2, Wl2, b2s, Wf1, bf1.reshape(1, HF),
      Wf2, bf2.reshape(1, D))
    return out


# speedup vs baseline: 1.5408x; 1.5408x over previous
import jax
import jax.numpy as jnp
from jax.experimental import pallas as pl
from jax.experimental.pallas import tpu as pltpu

B, S, D = 2, 2048, 1024
HR, HE, HF = 256, 4096, 1024

BS = 1024
NS = S // BS
HEB = 512
NK = HE // HEB


def _router_kernel(x_ref, wr1_ref, br1_ref, wdom_ref, wgate_ref,
                   dlog_ref, glog_ref):
    pooled = jnp.mean(x_ref[...], axis=1)
    h = jnp.tanh(
        jnp.dot(pooled, wr1_ref[...], preferred_element_type=jnp.float32)
        + br1_ref[...])
    dlog_ref[...] = jnp.dot(h, wdom_ref[...],
                            preferred_element_type=jnp.float32)
    glog_ref[...] = jnp.dot(h, wgate_ref[...],
                            preferred_element_type=jnp.float32)


def _expert_kernel(dom_ref, conf_ref, x_ref, w1m_ref, w1l_ref, b1_ref,
                   w2m_ref, w2l_ref, b2_ref, wf1_ref, bf1_ref, wf2_ref,
                   bf2_ref, o_ref, acc_ref):
    b = pl.program_id(0)
    k = pl.program_id(2)
    d = dom_ref[b]
    xb = x_ref[0]

    def _expert_step(w1_ref, w2_ref):
        h = jax.nn.gelu(
            jnp.dot(xb, w1_ref[...], preferred_element_type=jnp.float32)
            + b1_ref[0])
        part = jnp.dot(h, w2_ref[...], preferred_element_type=jnp.float32)

        @pl.when(k == 0)
        def _():
            acc_ref[...] = part

        @pl.when(k > 0)
        def _():
            acc_ref[...] += part

    @pl.when(d == 0)
    def _():
        _expert_step(w1m_ref, w2m_ref)

    @pl.when(d != 0)
    def _():
        _expert_step(w1l_ref, w2l_ref)

    @pl.when(k == NK - 1)
    def _():
        e = acc_ref[...] + b2_ref[0]
        t = jnp.tanh(
            jnp.dot(e, wf1_ref[...], preferred_element_type=jnp.float32)
            + bf1_ref[...])
        f = jnp.dot(t, wf2_ref[...],
                    preferred_element_type=jnp.float32) + bf2_ref[...]
        o_ref[0] = conf_ref[b] * f + xb


@jax.jit
def kernel(x, Wr1, br1, Wdom, Wop, Wtask, Wgate, Wm1, bm1, Wm2, bm2,
           Wl1, bl1, Wl2, bl2, Wf1, bf1, Wf2, bf2):
    del Wop, Wtask

    dlog, glog = pl.pallas_call(
        _router_kernel,
        out_shape=(
            jax.ShapeDtypeStruct((B, 2), jnp.float32),
            jax.ShapeDtypeStruct((B, 2), jnp.float32),
        ),
    )(x, Wr1, br1.reshape(1, HR), Wdom, Wgate)

    dom = (dlog[:, 1] > dlog[:, 0]).astype(jnp.int32)
    gmax = jnp.max(glog, axis=1, keepdims=True)
    eg = jnp.exp(glog - gmax)
    conf = jnp.take_along_axis(eg, dom[:, None], axis=1)[:, 0] / jnp.sum(eg, axis=1)

    b1s = jnp.stack([bm1, bl1]).reshape(2, 1, HE)
    b2s = jnp.stack([bm2, bl2]).reshape(2, 1, D)

    def w1_idx(sel):
        def idx(b, s, k, dom, conf):
            return (0, jnp.where(dom[b] == sel, k, 0))
        return idx

    def w2_idx(sel):
        def idx(b, s, k, dom, conf):
            return (jnp.where(dom[b] == sel, k, 0), 0)
        return idx

    grid = (B, NS, NK)
    out = pl.pallas_call(
        _expert_kernel,
        grid_spec=pltpu.PrefetchScalarGridSpec(
            num_scalar_prefetch=2,
            grid=grid,
            in_specs=[
                pl.BlockSpec((1, BS, D), lambda b, s, k, dom, conf: (b, s, 0)),
                pl.BlockSpec((D, HEB), w1_idx(0)),
                pl.BlockSpec((D, HEB), w1_idx(1)),
                pl.BlockSpec((1, 1, HEB),
                             lambda b, s, k, dom, conf: (dom[b], 0, k)),
                pl.BlockSpec((HEB, D), w2_idx(0)),
                pl.BlockSpec((HEB, D), w2_idx(1)),
                pl.BlockSpec((1, 1, D),
                             lambda b, s, k, dom, conf: (dom[b], 0, 0)),
                pl.BlockSpec((D, HF), lambda b, s, k, dom, conf: (0, 0)),
                pl.BlockSpec((1, HF), lambda b, s, k, dom, conf: (0, 0)),
                pl.BlockSpec((HF, D), lambda b, s, k, dom, conf: (0, 0)),
                pl.BlockSpec((1, D), lambda b, s, k, dom, conf: (0, 0)),
            ],
            out_specs=pl.BlockSpec((1, BS, D),
                                   lambda b, s, k, dom, conf: (b, s, 0)),
            scratch_shapes=[pltpu.VMEM((BS, D), jnp.float32)],
        ),
        out_shape=jax.ShapeDtypeStruct((B, S, D), jnp.float32),
    )(dom, conf, x, Wm1, Wl1, b1s, Wm2, Wl2, b2s, Wf1, bf1.reshape(1, HF),
      Wf2, bf2.reshape(1, D))
    return out


# speedup vs baseline: 1.6522x; 1.0723x over previous
import jax
import jax.numpy as jnp
from jax.experimental import pallas as pl
from jax.experimental.pallas import tpu as pltpu

B, S, D = 2, 2048, 1024
HR, HE, HF = 256, 4096, 1024

BS = 512
NS = S // BS
HEB = 1024
NK = HE // HEB


def _router_kernel(x_ref, wr1_ref, br1_ref, wdom_ref, wgate_ref,
                   dom_ref, conf_ref):
    pooled = jnp.mean(x_ref[...], axis=1)
    h = jnp.tanh(
        jnp.dot(pooled, wr1_ref[...], preferred_element_type=jnp.float32)
        + br1_ref[...])
    dlog = jnp.dot(h, wdom_ref[...],
                   preferred_element_type=jnp.float32)
    glog = jnp.dot(h, wgate_ref[...],
                   preferred_element_type=jnp.float32)
    sel1 = dlog[:, 1:2] > dlog[:, 0:1]
    dom_ref[...] = sel1.astype(jnp.int32)
    g0, g1 = glog[:, 0:1], glog[:, 1:2]
    m = jnp.maximum(g0, g1)
    e0 = jnp.exp(g0 - m)
    e1 = jnp.exp(g1 - m)
    conf_ref[...] = jnp.where(sel1, e1, e0) / (e0 + e1)


def _expert_kernel(dom_ref, conf_ref, x_ref, w1m_ref, w1l_ref, b1m_ref,
                   b1l_ref, w2m_ref, w2l_ref, b2m_ref, b2l_ref, wf1_ref,
                   bf1_ref, wf2_ref, bf2_ref, o_ref, acc_ref):
    b = pl.program_id(0)
    k = pl.program_id(2)
    d = dom_ref[b, 0]
    xb = x_ref[0]

    def _expert_step(w1_ref, b1_ref, w2_ref):
        h = jax.nn.gelu(
            jnp.dot(xb, w1_ref[...], preferred_element_type=jnp.float32)
            + b1_ref[...])
        part = jnp.dot(h, w2_ref[...], preferred_element_type=jnp.float32)

        @pl.when(k == 0)
        def _():
            acc_ref[...] = part

        @pl.when(k > 0)
        def _():
            acc_ref[...] += part

    @pl.when(d == 0)
    def _():
        _expert_step(w1m_ref, b1m_ref, w2m_ref)

    @pl.when(d != 0)
    def _():
        _expert_step(w1l_ref, b1l_ref, w2l_ref)

    @pl.when(k == NK - 1)
    def _():
        b2 = jnp.where(d == 0, b2m_ref[...], b2l_ref[...])
        e = acc_ref[...] + b2
        t = jnp.tanh(
            jnp.dot(e, wf1_ref[...], preferred_element_type=jnp.float32)
            + bf1_ref[...])
        f = jnp.dot(t, wf2_ref[...],
                    preferred_element_type=jnp.float32) + bf2_ref[...]
        o_ref[0] = conf_ref[b, 0] * f + xb


@jax.jit
def kernel(x, Wr1, br1, Wdom, Wop, Wtask, Wgate, Wm1, bm1, Wm2, bm2,
           Wl1, bl1, Wl2, bl2, Wf1, bf1, Wf2, bf2):
    del Wop, Wtask

    dom, conf = pl.pallas_call(
        _router_kernel,
        out_shape=(
            jax.ShapeDtypeStruct((B, 1), jnp.int32),
            jax.ShapeDtypeStruct((B, 1), jnp.float32),
        ),
    )(x, Wr1, br1.reshape(1, HR), Wdom, Wgate)

    def w1_idx(sel):
        def idx(b, s, k, dom, conf):
            return (0, jnp.where(dom[b, 0] == sel, k, 0))
        return idx

    def w2_idx(sel):
        def idx(b, s, k, dom, conf):
            return (jnp.where(dom[b, 0] == sel, k, 0), 0)
        return idx

    grid = (B, NS, NK)
    out = pl.pallas_call(
        _expert_kernel,
        grid_spec=pltpu.PrefetchScalarGridSpec(
            num_scalar_prefetch=2,
            grid=grid,
            in_specs=[
                pl.BlockSpec((1, BS, D), lambda b, s, k, dom, conf: (b, s, 0)),
                pl.BlockSpec((D, HEB), w1_idx(0)),
                pl.BlockSpec((D, HEB), w1_idx(1)),
                pl.BlockSpec((1, HEB), w1_idx(0)),
                pl.BlockSpec((1, HEB), w1_idx(1)),
                pl.BlockSpec((HEB, D), w2_idx(0)),
                pl.BlockSpec((HEB, D), w2_idx(1)),
                pl.BlockSpec((1, D), lambda b, s, k, dom, conf: (0, 0)),
                pl.BlockSpec((1, D), lambda b, s, k, dom, conf: (0, 0)),
                pl.BlockSpec((D, HF), lambda b, s, k, dom, conf: (0, 0)),
                pl.BlockSpec((1, HF), lambda b, s, k, dom, conf: (0, 0)),
                pl.BlockSpec((HF, D), lambda b, s, k, dom, conf: (0, 0)),
                pl.BlockSpec((1, D), lambda b, s, k, dom, conf: (0, 0)),
            ],
            out_specs=pl.BlockSpec((1, BS, D),
                                   lambda b, s, k, dom, conf: (b, s, 0)),
            scratch_shapes=[pltpu.VMEM((BS, D), jnp.float32)],
        ),
        out_shape=jax.ShapeDtypeStruct((B, S, D), jnp.float32),
    )(dom, conf, x, Wm1, Wl1, bm1.reshape(1, HE), bl1.reshape(1, HE),
      Wm2, Wl2, bm2.reshape(1, D), bl2.reshape(1, D), Wf1,
      bf1.reshape(1, HF), Wf2, bf2.reshape(1, D))
    return out


# speedup vs baseline: 1.6562x; 1.0024x over previous
import jax
import jax.numpy as jnp
from jax.experimental import pallas as pl
from jax.experimental.pallas import tpu as pltpu

B, S, D = 2, 2048, 1024
HR, HE, HF = 256, 4096, 1024

BS = 512
NS = S // BS
HEB = 1024
NK = HE // HEB


def _router_kernel(x_ref, wr1_ref, br1_ref, wdom_ref, wgate_ref,
                   dom_ref, conf_ref):
    pooled = jnp.mean(x_ref[...], axis=1)
    h = jnp.tanh(
        jnp.dot(pooled, wr1_ref[...], preferred_element_type=jnp.float32)
        + br1_ref[...])
    dlog = jnp.dot(h, wdom_ref[...],
                   preferred_element_type=jnp.float32)
    glog = jnp.dot(h, wgate_ref[...],
                   preferred_element_type=jnp.float32)
    sel1 = dlog[:, 1:2] > dlog[:, 0:1]
    dom_ref[...] = sel1.astype(jnp.int32)
    g0, g1 = glog[:, 0:1], glog[:, 1:2]
    m = jnp.maximum(g0, g1)
    e0 = jnp.exp(g0 - m)
    e1 = jnp.exp(g1 - m)
    conf_ref[...] = jnp.where(sel1, e1, e0) / (e0 + e1)


def _expert_kernel(dom_ref, conf_ref, x_ref, w1m_ref, w1l_ref, b1m_ref,
                   b1l_ref, w2m_ref, w2l_ref, b2m_ref, b2l_ref, wf1_ref,
                   bf1_ref, wf2_ref, bf2_ref, o_ref, acc_ref):
    b = pl.program_id(0)
    k = pl.program_id(2)
    d = dom_ref[b, 0]
    xb = x_ref[0]

    def _expert_step(w1_ref, b1_ref, w2_ref):
        h = jax.nn.gelu(
            jnp.dot(xb, w1_ref[...], preferred_element_type=jnp.float32)
            + b1_ref[...])
        part = jnp.dot(h, w2_ref[...], preferred_element_type=jnp.float32)

        @pl.when(k == 0)
        def _():
            acc_ref[...] = part

        @pl.when(k > 0)
        def _():
            acc_ref[...] += part

    @pl.when(d == 0)
    def _():
        _expert_step(w1m_ref, b1m_ref, w2m_ref)

    @pl.when(d != 0)
    def _():
        _expert_step(w1l_ref, b1l_ref, w2l_ref)

    @pl.when(k == NK - 1)
    def _():
        b2 = jnp.where(d == 0, b2m_ref[...], b2l_ref[...])
        e = acc_ref[...] + b2
        t = jnp.tanh(
            jnp.dot(e, wf1_ref[...], preferred_element_type=jnp.float32)
            + bf1_ref[...])
        f = jnp.dot(t, wf2_ref[...],
                    preferred_element_type=jnp.float32) + bf2_ref[...]
        o_ref[0] = conf_ref[b, 0] * f + xb


@jax.jit
def kernel(x, Wr1, br1, Wdom, Wop, Wtask, Wgate, Wm1, bm1, Wm2, bm2,
           Wl1, bl1, Wl2, bl2, Wf1, bf1, Wf2, bf2):
    del Wop, Wtask

    dom, conf = pl.pallas_call(
        _router_kernel,
        out_shape=(
            jax.ShapeDtypeStruct((B, 1), jnp.int32),
            jax.ShapeDtypeStruct((B, 1), jnp.float32),
        ),
    )(x, Wr1, br1.reshape(1, HR), Wdom, Wgate)

    def w1_idx(sel):
        def idx(b, s, k, dom, conf):
            return (0, jnp.where(dom[b, 0] == sel, k, 0))
        return idx

    def w2_idx(sel):
        def idx(b, s, k, dom, conf):
            return (jnp.where(dom[b, 0] == sel, k, 0), 0)
        return idx

    grid = (B, NS, NK)
    out = pl.pallas_call(
        _expert_kernel,
        grid_spec=pltpu.PrefetchScalarGridSpec(
            num_scalar_prefetch=2,
            grid=grid,
            in_specs=[
                pl.BlockSpec((1, BS, D), lambda b, s, k, dom, conf: (b, s, 0)),
                pl.BlockSpec((D, HEB), w1_idx(0)),
                pl.BlockSpec((D, HEB), w1_idx(1)),
                pl.BlockSpec((1, HEB), w1_idx(0)),
                pl.BlockSpec((1, HEB), w1_idx(1)),
                pl.BlockSpec((HEB, D), w2_idx(0)),
                pl.BlockSpec((HEB, D), w2_idx(1)),
                pl.BlockSpec((1, D), lambda b, s, k, dom, conf: (0, 0)),
                pl.BlockSpec((1, D), lambda b, s, k, dom, conf: (0, 0)),
                pl.BlockSpec((D, HF), lambda b, s, k, dom, conf: (0, 0)),
                pl.BlockSpec((1, HF), lambda b, s, k, dom, conf: (0, 0)),
                pl.BlockSpec((HF, D), lambda b, s, k, dom, conf: (0, 0)),
                pl.BlockSpec((1, D), lambda b, s, k, dom, conf: (0, 0)),
            ],
            out_specs=pl.BlockSpec((1, BS, D),
                                   lambda b, s, k, dom, conf: (b, s, 0)),
            scratch_shapes=[pltpu.VMEM((BS, D), jnp.float32)],
        ),
        compiler_params=pltpu.CompilerParams(
            dimension_semantics=("parallel", "parallel", "arbitrary")),
        out_shape=jax.ShapeDtypeStruct((B, S, D), jnp.float32),
    )(dom, conf, x, Wm1, Wl1, bm1.reshape(1, HE), bl1.reshape(1, HE),
      Wm2, Wl2, bm2.reshape(1, D), bl2.reshape(1, D), Wf1,
      bf1.reshape(1, HF), Wf2, bf2.reshape(1, D))
    return out
